# Initial kernel scaffold; baseline (speedup 1.0000x reference)
import functools

import jax
import jax.numpy as jnp
from jax import lax
from jax.experimental import pallas as pl
from jax.experimental.pallas import tpu as pltpu
from jax.experimental.pallas import tpu_sc as plsc

POOL = 7
NSAMP = POOL * POOL
NC, NS, LANES = 2, 16, 16
NW = NC * NS


def _log2(x):
    return jnp.log(x) / jnp.log(2.0)


def _prep(boxes, image_shape, sizes):
    f32 = jnp.float32
    N = boxes.shape[0] * boxes.shape[1]
    fb = boxes.reshape(N, 4)
    y1 = fb[:, 0]
    x1 = fb[:, 1]
    y2 = fb[:, 2]
    x2 = fb[:, 3]
    h = y2 - y1
    w = x2 - x1
    image_area = (image_shape[0] * image_shape[1]).astype(f32)
    roi_level = _log2(jnp.sqrt(h * w) / (224.0 / jnp.sqrt(image_area)))
    roi_level = jnp.minimum(
        5, jnp.maximum(2, 4 + jnp.round(roi_level).astype(jnp.int32))
    )

    bases = []
    acc = 0
    for H in sizes:
        bases.append(acc)
        acc += H * H

    ar = jnp.arange(POOL, dtype=f32)[None, :]
    sel_ly = jnp.zeros((N, POOL), f32)
    sel_lx = jnp.zeros((N, POOL), f32)
    sel_ry0 = jnp.zeros((N, POOL), jnp.int32)
    sel_ry1 = jnp.zeros((N, POOL), jnp.int32)
    sel_cx0 = jnp.zeros((N, POOL), jnp.int32)
    sel_cx1 = jnp.zeros((N, POOL), jnp.int32)
    for li, H in enumerate(sizes):
        level = li + 2
        W = H
        ys = y1[:, None] * (H - 1) + ar * ((y2 - y1)[:, None] * (H - 1) / (POOL - 1))
        xs = x1[:, None] * (W - 1) + ar * ((x2 - x1)[:, None] * (W - 1) / (POOL - 1))
        y0f = jnp.floor(ys)
        x0f = jnp.floor(xs)
        y0 = jnp.clip(y0f.astype(jnp.int32), 0, H - 1)
        y1i = jnp.clip(y0 + 1, 0, H - 1)
        x0 = jnp.clip(x0f.astype(jnp.int32), 0, W - 1)
        x1c = jnp.clip(x0 + 1, 0, W - 1)
        ly = ys - y0f
        lx = xs - x0f
        m = (roi_level == level)[:, None]
        sel_ly = jnp.where(m, ly, sel_ly)
        sel_lx = jnp.where(m, lx, sel_lx)
        sel_ry0 = jnp.where(m, bases[li] + y0 * W, sel_ry0)
        sel_ry1 = jnp.where(m, bases[li] + y1i * W, sel_ry1)
        sel_cx0 = jnp.where(m, x0, sel_cx0)
        sel_cx1 = jnp.where(m, x1c, sel_cx1)

    def cross(ry, cx):
        return (ry[:, :, None] + cx[:, None, :]).reshape(N, NSAMP)

    idx4 = jnp.stack(
        [cross(sel_ry0, sel_cx0), cross(sel_ry0, sel_cx1),
         cross(sel_ry1, sel_cx0), cross(sel_ry1, sel_cx1)], axis=1
    )
    wy = jnp.broadcast_to(sel_ly[:, :, None, None], (N, POOL, POOL, LANES))
    wy = wy.reshape(N, NSAMP, LANES)
    wx = jnp.broadcast_to(sel_lx[:, None, :, None], (N, POOL, POOL, LANES))
    wx = wx.reshape(N, NSAMP, LANES)
    return idx4, wy, wx


def _make_sc_call(N, C):
    CCH = C // LANES
    q, r = divmod(N, NW)
    mesh = plsc.VectorSubcoreMesh(
        core_axis_name="c", subcore_axis_name="s", num_cores=NC, num_subcores=NS
    )

    @functools.partial(
        pl.kernel,
        out_type=jax.ShapeDtypeStruct((N * NSAMP, C), jnp.float32),
        mesh=mesh,
        scratch_types=[
            pltpu.VMEM((4, NSAMP), jnp.int32),
            pltpu.VMEM((NSAMP, LANES), jnp.float32),
            pltpu.VMEM((NSAMP, LANES), jnp.float32),
            pltpu.VMEM((4, NSAMP, C), jnp.float32),
            pltpu.VMEM((NSAMP, C), jnp.float32),
            pltpu.SemaphoreType.DMA,
        ],
    )
    def roialign_sc(table, idx_hbm, wy_hbm, wx_hbm, out_hbm,
                    idx_v, wy_v, wx_v, rows_v, out_v, sem):
        wid = lax.axis_index("s") * NC + lax.axis_index("c")
        nb = jnp.where(wid < r, q + 1, q)
        start = wid * q + jnp.minimum(wid, r)

        def box_body(b, carry):
            n = start + b
            pltpu.sync_copy(idx_hbm.at[n], idx_v)
            pltpu.sync_copy(wy_hbm.at[n], wy_v)
            pltpu.sync_copy(wx_hbm.at[n], wx_v)
            cps = [
                pltpu.async_copy(table.at[idx_v.at[k]], rows_v.at[k], sem)
                for k in range(4)
            ]
            for cp in cps:
                cp.wait()

            def samp(j, c2):
                lyv = wy_v[j]
                lxv = wx_v[j]
                for c in range(CCH):
                    s = pl.ds(c * LANES, LANES)
                    v00 = rows_v[0, j, s]
                    v01 = rows_v[1, j, s]
                    v10 = rows_v[2, j, s]
                    v11 = rows_v[3, j, s]
                    top = v00 + (v01 - v00) * lxv
                    bot = v10 + (v11 - v10) * lxv
                    out_v[j, s] = top + (bot - top) * lyv
                return c2

            lax.fori_loop(0, NSAMP, samp, 0)
            pltpu.sync_copy(out_v, out_hbm.at[pl.ds(n * NSAMP, NSAMP)])
            return carry

        lax.fori_loop(0, nb, box_body, 0)

    return roialign_sc


def kernel(boxes, image_shape, P2, P3, P4, P5):
    B, N = boxes.shape[0], boxes.shape[1]
    C = P2.shape[-1]
    sizes = (P2.shape[1], P3.shape[1], P4.shape[1], P5.shape[1])
    idx4, wy, wx = _prep(boxes, image_shape, sizes)
    table = jnp.concatenate(
        [p.reshape(-1, C) for p in (P2, P3, P4, P5)], axis=0
    )
    out = _make_sc_call(B * N, C)(table, idx4, wy, wx)
    return out.reshape(B, N, POOL, POOL, C)


# speedup vs baseline: 12.4311x; 12.4311x over previous
import functools

import jax
import jax.numpy as jnp
from jax import lax
from jax.experimental import pallas as pl
from jax.experimental.pallas import tpu as pltpu
from jax.experimental.pallas import tpu_sc as plsc

POOL = 7
NSAMP = POOL * POOL
NC, NS, LANES = 2, 16, 16
NW = NC * NS


def _log2(x):
    return jnp.log(x) / jnp.log(2.0)


def _prep(boxes, image_shape, sizes):
    f32 = jnp.float32
    N = boxes.shape[0] * boxes.shape[1]
    fb = boxes.reshape(N, 4)
    y1 = fb[:, 0]
    x1 = fb[:, 1]
    y2 = fb[:, 2]
    x2 = fb[:, 3]
    h = y2 - y1
    w = x2 - x1
    image_area = (image_shape[0] * image_shape[1]).astype(f32)
    roi_level = _log2(jnp.sqrt(h * w) / (224.0 / jnp.sqrt(image_area)))
    roi_level = jnp.minimum(
        5, jnp.maximum(2, 4 + jnp.round(roi_level).astype(jnp.int32))
    )

    bases = []
    acc = 0
    for H in sizes:
        bases.append(acc)
        acc += H * H

    ar = jnp.arange(POOL, dtype=f32)[None, :]
    sel_ly = jnp.zeros((N, POOL), f32)
    sel_lx = jnp.zeros((N, POOL), f32)
    sel_ry0 = jnp.zeros((N, POOL), jnp.int32)
    sel_ry1 = jnp.zeros((N, POOL), jnp.int32)
    sel_cx0 = jnp.zeros((N, POOL), jnp.int32)
    sel_cx1 = jnp.zeros((N, POOL), jnp.int32)
    for li, H in enumerate(sizes):
        level = li + 2
        W = H
        ys = y1[:, None] * (H - 1) + ar * ((y2 - y1)[:, None] * (H - 1) / (POOL - 1))
        xs = x1[:, None] * (W - 1) + ar * ((x2 - x1)[:, None] * (W - 1) / (POOL - 1))
        y0f = jnp.floor(ys)
        x0f = jnp.floor(xs)
        y0 = jnp.clip(y0f.astype(jnp.int32), 0, H - 1)
        y1i = jnp.clip(y0 + 1, 0, H - 1)
        x0 = jnp.clip(x0f.astype(jnp.int32), 0, W - 1)
        x1c = jnp.clip(x0 + 1, 0, W - 1)
        ly = ys - y0f
        lx = xs - x0f
        m = (roi_level == level)[:, None]
        sel_ly = jnp.where(m, ly, sel_ly)
        sel_lx = jnp.where(m, lx, sel_lx)
        sel_ry0 = jnp.where(m, bases[li] + y0 * W, sel_ry0)
        sel_ry1 = jnp.where(m, bases[li] + y1i * W, sel_ry1)
        sel_cx0 = jnp.where(m, x0, sel_cx0)
        sel_cx1 = jnp.where(m, x1c, sel_cx1)

    def cross(ry, cx):
        return (ry[:, :, None] + cx[:, None, :]).reshape(N, NSAMP)

    idx4 = jnp.stack(
        [cross(sel_ry0, sel_cx0), cross(sel_ry0, sel_cx1),
         cross(sel_ry1, sel_cx0), cross(sel_ry1, sel_cx1)], axis=1
    )
    wy = jnp.broadcast_to(sel_ly[:, :, None, None], (N, POOL, POOL, LANES))
    wy = wy.reshape(N, NSAMP, LANES)
    wx = jnp.broadcast_to(sel_lx[:, None, :, None], (N, POOL, POOL, LANES))
    wx = wx.reshape(N, NSAMP, LANES)
    return idx4, wy, wx


def _make_sc_call(N, C):
    CCH = C // LANES
    q, r = divmod(N, NW)
    mesh = plsc.VectorSubcoreMesh(
        core_axis_name="c", subcore_axis_name="s", num_cores=NC, num_subcores=NS
    )

    @functools.partial(
        pl.kernel,
        out_type=jax.ShapeDtypeStruct((N * NSAMP, C), jnp.float32),
        mesh=mesh,
        compiler_params=pltpu.CompilerParams(use_tc_tiling_on_sc=False),
        scratch_types=[
            pltpu.VMEM((4, NSAMP), jnp.int32),
            pltpu.VMEM((NSAMP, LANES), jnp.float32),
            pltpu.VMEM((NSAMP, LANES), jnp.float32),
            pltpu.VMEM((4, NSAMP, C), jnp.float32),
            pltpu.VMEM((NSAMP, C), jnp.float32),
            pltpu.SemaphoreType.DMA,
        ],
    )
    def roialign_sc(table, idx_hbm, wy_hbm, wx_hbm, out_hbm,
                    idx_v, wy_v, wx_v, rows_v, out_v, sem):
        wid = lax.axis_index("s") * NC + lax.axis_index("c")
        nb = jnp.where(wid < r, q + 1, q)
        start = wid * q + jnp.minimum(wid, r)

        def box_body(b, carry):
            n = start + b
            pltpu.sync_copy(idx_hbm.at[n], idx_v)
            pltpu.sync_copy(wy_hbm.at[n], wy_v)
            pltpu.sync_copy(wx_hbm.at[n], wx_v)
            cps = [
                pltpu.async_copy(table.at[idx_v.at[k]], rows_v.at[k], sem)
                for k in range(4)
            ]
            for cp in cps:
                cp.wait()

            def samp(j, c2):
                lyv = wy_v[j]
                lxv = wx_v[j]
                for c in range(CCH):
                    s = pl.ds(c * LANES, LANES)
                    v00 = rows_v[0, j, s]
                    v01 = rows_v[1, j, s]
                    v10 = rows_v[2, j, s]
                    v11 = rows_v[3, j, s]
                    top = v00 + (v01 - v00) * lxv
                    bot = v10 + (v11 - v10) * lxv
                    out_v[j, s] = top + (bot - top) * lyv
                return c2

            lax.fori_loop(0, NSAMP, samp, 0)
            pltpu.sync_copy(out_v, out_hbm.at[pl.ds(n * NSAMP, NSAMP)])
            return carry

        lax.fori_loop(0, nb, box_body, 0)

    return roialign_sc


def kernel(boxes, image_shape, P2, P3, P4, P5):
    B, N = boxes.shape[0], boxes.shape[1]
    C = P2.shape[-1]
    sizes = (P2.shape[1], P3.shape[1], P4.shape[1], P5.shape[1])
    idx4, wy, wx = _prep(boxes, image_shape, sizes)
    table = jnp.concatenate(
        [p.reshape(-1, C) for p in (P2, P3, P4, P5)], axis=0
    )
    out = _make_sc_call(B * N, C)(table, idx4, wy, wx)
    return out.reshape(B, N, POOL, POOL, C)


# speedup vs baseline: 13.6570x; 1.0986x over previous
import functools

import jax
import jax.numpy as jnp
from jax import lax
from jax.experimental import pallas as pl
from jax.experimental.pallas import tpu as pltpu
from jax.experimental.pallas import tpu_sc as plsc

POOL = 7
NSAMP = POOL * POOL
NC, NS, LANES = 2, 16, 16
NW = NC * NS


def _log2(x):
    return jnp.log(x) / jnp.log(2.0)


def _prep(boxes, image_shape, sizes):
    f32 = jnp.float32
    N = boxes.shape[0] * boxes.shape[1]
    fb = boxes.reshape(N, 4)
    y1 = fb[:, 0]
    x1 = fb[:, 1]
    y2 = fb[:, 2]
    x2 = fb[:, 3]
    h = y2 - y1
    w = x2 - x1
    image_area = (image_shape[0] * image_shape[1]).astype(f32)
    roi_level = _log2(jnp.sqrt(h * w) / (224.0 / jnp.sqrt(image_area)))
    roi_level = jnp.minimum(
        5, jnp.maximum(2, 4 + jnp.round(roi_level).astype(jnp.int32))
    )

    ar = jnp.arange(POOL, dtype=f32)[None, :]
    sel_ly = jnp.zeros((N, POOL), f32)
    sel_lx = jnp.zeros((N, POOL), f32)
    sel_ry0 = jnp.zeros((N, POOL), jnp.int32)
    sel_ry1 = jnp.zeros((N, POOL), jnp.int32)
    sel_cx0 = jnp.zeros((N, POOL), jnp.int32)
    sel_cx1 = jnp.zeros((N, POOL), jnp.int32)
    for li, H in enumerate(sizes):
        level = li + 2
        W = H
        ys = y1[:, None] * (H - 1) + ar * ((y2 - y1)[:, None] * (H - 1) / (POOL - 1))
        xs = x1[:, None] * (W - 1) + ar * ((x2 - x1)[:, None] * (W - 1) / (POOL - 1))
        y0f = jnp.floor(ys)
        x0f = jnp.floor(xs)
        y0 = jnp.clip(y0f.astype(jnp.int32), 0, H - 1)
        y1i = jnp.clip(y0 + 1, 0, H - 1)
        x0 = jnp.clip(x0f.astype(jnp.int32), 0, W - 1)
        x1c = jnp.clip(x0 + 1, 0, W - 1)
        ly = ys - y0f
        lx = xs - x0f
        m = (roi_level == level)[:, None]
        sel_ly = jnp.where(m, ly, sel_ly)
        sel_lx = jnp.where(m, lx, sel_lx)
        sel_ry0 = jnp.where(m, y0 * W, sel_ry0)
        sel_ry1 = jnp.where(m, y1i * W, sel_ry1)
        sel_cx0 = jnp.where(m, x0, sel_cx0)
        sel_cx1 = jnp.where(m, x1c, sel_cx1)

    def cross(ry, cx):
        return (ry[:, :, None] + cx[:, None, :]).reshape(N, NSAMP)

    idx4 = jnp.stack(
        [cross(sel_ry0, sel_cx0), cross(sel_ry0, sel_cx1),
         cross(sel_ry1, sel_cx0), cross(sel_ry1, sel_cx1)], axis=1
    )
    wy = jnp.broadcast_to(sel_ly[:, :, None, None], (N, POOL, POOL, LANES))
    wy = wy.reshape(N, NSAMP, LANES)
    wx = jnp.broadcast_to(sel_lx[:, None, :, None], (N, POOL, POOL, LANES))
    wx = wx.reshape(N, NSAMP, LANES)
    return idx4, wy, wx, roi_level


def _make_sc_call(N, C):
    CCH = C // LANES
    BPW = (N + NW - 1) // NW
    mesh = plsc.VectorSubcoreMesh(
        core_axis_name="c", subcore_axis_name="s", num_cores=NC, num_subcores=NS
    )

    @functools.partial(
        pl.kernel,
        out_type=jax.ShapeDtypeStruct((N * NSAMP, C), jnp.float32),
        mesh=mesh,
        compiler_params=pltpu.CompilerParams(use_tc_tiling_on_sc=False),
        scratch_types=[
            pltpu.VMEM((BPW + LANES,), jnp.int32),
            pltpu.VMEM((4, NSAMP), jnp.int32),
            pltpu.VMEM((NSAMP, LANES), jnp.float32),
            pltpu.VMEM((NSAMP, LANES), jnp.float32),
            pltpu.VMEM((4, NSAMP, C), jnp.float32),
            pltpu.VMEM((NSAMP, C), jnp.float32),
            pltpu.SemaphoreType.DMA,
        ],
    )
    def roialign_sc(t2, t3, t4, t5, idx_hbm, wy_hbm, wx_hbm, lvl_hbm, out_hbm,
                    lvl_v, idx_v, wy_v, wx_v, rows_v, out_v, sem):
        wid = lax.axis_index("s") * NC + lax.axis_index("c")
        start = wid * BPW
        nb = jnp.clip(N - start, 0, BPW)
        pltpu.sync_copy(lvl_hbm.at[pl.ds(start, BPW + LANES)], lvl_v)

        def box_body(b, carry):
            n = start + b
            pltpu.sync_copy(idx_hbm.at[n], idx_v)
            pltpu.sync_copy(wy_hbm.at[n], wy_v)
            pltpu.sync_copy(wx_hbm.at[n], wx_v)
            lvl = lvl_v[pl.ds(b, LANES)][0]
            for level, tref in ((2, t2), (3, t3), (4, t4), (5, t5)):
                def gather(tref=tref):
                    cps = [
                        pltpu.async_copy(tref.at[idx_v.at[k]], rows_v.at[k], sem)
                        for k in range(4)
                    ]
                    for cp in cps:
                        cp.wait()
                pl.when(lvl == level)(gather)

            def samp(j, c2):
                lyv = wy_v[j]
                lxv = wx_v[j]
                for c in range(CCH):
                    s = pl.ds(c * LANES, LANES)
                    v00 = rows_v[0, j, s]
                    v01 = rows_v[1, j, s]
                    v10 = rows_v[2, j, s]
                    v11 = rows_v[3, j, s]
                    top = v00 + (v01 - v00) * lxv
                    bot = v10 + (v11 - v10) * lxv
                    out_v[j, s] = top + (bot - top) * lyv
                return c2

            lax.fori_loop(0, NSAMP, samp, 0)
            pltpu.sync_copy(out_v, out_hbm.at[pl.ds(n * NSAMP, NSAMP)])
            return carry

        lax.fori_loop(0, nb, box_body, 0)

    return roialign_sc


def kernel(boxes, image_shape, P2, P3, P4, P5):
    B, N = boxes.shape[0], boxes.shape[1]
    C = P2.shape[-1]
    sizes = (P2.shape[1], P3.shape[1], P4.shape[1], P5.shape[1])
    idx4, wy, wx, lvl = _prep(boxes, image_shape, sizes)
    BPW = (B * N + NW - 1) // NW
    lvl_pad = jnp.pad(lvl, (0, NW * BPW + LANES - B * N), constant_values=2)
    tables = [p.reshape(-1, C) for p in (P2, P3, P4, P5)]
    out = _make_sc_call(B * N, C)(*tables, idx4, wy, wx, lvl_pad)
    return out.reshape(B, N, POOL, POOL, C)
